# baseline (speedup 1.0000x reference)
import functools

import jax
import jax.numpy as jnp
from jax import lax
from jax.experimental import pallas as pl
from jax.experimental.pallas import tpu as pltpu
from jax.experimental.pallas import tpu_sc as plsc

D = 64
L = 16
EPS = 1e-12
CHUNK = 128


def _rsqrt(x):
    i = plsc.bitcast(x, jnp.int32)
    i = jnp.int32(0x5F3759DF) - (i >> 1)
    y = plsc.bitcast(i, jnp.float32)
    h = x * jnp.float32(0.5)
    for _ in range(3):
        y = y * (jnp.float32(1.5) - h * y * y)
    return y


@functools.partial(jax.jit, static_argnames=("b_total",))
def _embed_normalize(x_flat, table, b_total):
    info = plsc.get_sparse_core_info()
    nc, ns = info.num_cores, info.num_subcores
    nw = nc * ns
    b_per_w = b_total // nw
    n_chunks = b_per_w // CHUNK
    mesh = plsc.VectorSubcoreMesh(core_axis_name="c", subcore_axis_name="s")

    @functools.partial(
        pl.kernel,
        mesh=mesh,
        out_type=jax.ShapeDtypeStruct((b_total, D), jnp.float32),
        compiler_params=pltpu.CompilerParams(
            needs_layout_passes=False, use_tc_tiling_on_sc=False),
        scratch_types=[
            pltpu.VMEM((CHUNK,), jnp.int32),
            pltpu.VMEM((CHUNK, D), jnp.float32),
            pltpu.SemaphoreType.DMA,
        ],
    )
    def body(x_hbm, table_hbm, out_hbm, idx_v, rows_v, sem):
        wid = lax.axis_index("s") * nc + lax.axis_index("c")
        base = wid * b_per_w

        def chunk_body(g, carry):
            row0 = base + g * CHUNK
            pltpu.sync_copy(x_hbm.at[pl.ds(row0, CHUNK)], idx_v)
            pltpu.async_copy(table_hbm.at[idx_v], rows_v, sem).wait()

            def grp(t, c):
                row0 = t * L
                for r in range(L):
                    vs = [rows_v[row0 + r, pl.ds(q * L, L)]
                          for q in range(D // L)]
                    acc = None
                    for v in vs:
                        acc = v * v if acc is None else acc + v * v
                    sv = jnp.full((L,), jnp.sum(acc), jnp.float32)
                    inv = _rsqrt(sv)
                    nrm = sv * inv
                    scale = (jnp.float32(1.0)
                             / jnp.maximum(nrm, jnp.float32(EPS)))
                    for q, v in enumerate(vs):
                        rows_v[row0 + r, pl.ds(q * L, L)] = v * scale
                return c

            lax.fori_loop(0, CHUNK // L, grp, 0)
            pltpu.sync_copy(rows_v, out_hbm.at[pl.ds(row0, CHUNK)])
            return carry

        lax.fori_loop(0, n_chunks, chunk_body, 0)

    return body(x_flat, table)


def kernel(x, table):
    b, h = x.shape
    out = _embed_normalize(x.reshape(-1), table, b * h)
    return out.reshape(b, h, D)
